# Initial kernel scaffold; baseline (speedup 1.0000x reference)
import jax
import jax.numpy as jnp
from jax.experimental import pallas as pl


def kernel(x, edge_index, W1, att_src1, att_dst1, b1, W2, att_src2, att_dst2, b2):
    raise NotImplementedError("write your pallas kernel here")


# speedup vs baseline: 1.1214x; 1.1214x over previous
import functools
import jax
import jax.numpy as jnp
from jax.experimental import pallas as pl
from jax.experimental.pallas import tpu as pltpu

N = 10000
E = 160000
F_IN = 256
HID = 256
HEADS = 8
NCLS = 40

ROW_BLK = 400


def _mm1_body(x_ref, w_ref, a_ref, h_ref, ab_ref):
    h = jnp.dot(x_ref[...], w_ref[...], preferred_element_type=jnp.float32)
    h_ref[...] = h
    ab_ref[...] = jnp.dot(h, a_ref[...], preferred_element_type=jnp.float32)


def _layer1_matmul(x, W1, A1):
    grid = (N // ROW_BLK,)
    h, ab = pl.pallas_call(
        _mm1_body,
        grid=grid,
        in_specs=[
            pl.BlockSpec((ROW_BLK, F_IN), lambda i: (i, 0)),
            pl.BlockSpec((F_IN, HEADS * HID), lambda i: (0, 0)),
            pl.BlockSpec((HEADS * HID, 16), lambda i: (0, 0)),
        ],
        out_specs=[
            pl.BlockSpec((ROW_BLK, HEADS * HID), lambda i: (i, 0)),
            pl.BlockSpec((ROW_BLK, 16), lambda i: (i, 0)),
        ],
        out_shape=[
            jax.ShapeDtypeStruct((N, HEADS * HID), jnp.float32),
            jax.ShapeDtypeStruct((N, 16), jnp.float32),
        ],
    )(x, W1, A1)
    return h, ab


def _mm2_body(x_ref, w_ref, a_ref, h_ref, ab_ref):
    h = jnp.dot(x_ref[...], w_ref[...], preferred_element_type=jnp.float32)
    h_ref[...] = h
    ab_ref[...] = jnp.dot(h, a_ref[...], preferred_element_type=jnp.float32)


def _layer2_matmul(act, W2, A2):
    grid = (N // ROW_BLK,)
    h2, ab2 = pl.pallas_call(
        _mm2_body,
        grid=grid,
        in_specs=[
            pl.BlockSpec((ROW_BLK, HEADS * HID), lambda i: (i, 0)),
            pl.BlockSpec((HEADS * HID, NCLS), lambda i: (0, 0)),
            pl.BlockSpec((NCLS, 2), lambda i: (0, 0)),
        ],
        out_specs=[
            pl.BlockSpec((ROW_BLK, NCLS), lambda i: (i, 0)),
            pl.BlockSpec((ROW_BLK, 2), lambda i: (i, 0)),
        ],
        out_shape=[
            jax.ShapeDtypeStruct((N, NCLS), jnp.float32),
            jax.ShapeDtypeStruct((N, 2), jnp.float32),
        ],
    )(act, W2, A2)
    return h2, ab2


def _final_body(o_ref, d_ref, b_ref, out_ref):
    z = o_ref[...] / (d_ref[...] + 1e-16) + b_ref[...]
    m = jnp.max(z, axis=1, keepdims=True)
    s = jnp.sum(jnp.exp(z - m), axis=1, keepdims=True)
    out_ref[...] = z - m - jnp.log(s)


def _final_logsoftmax(out2, denom2, b2):
    grid = (N // ROW_BLK,)
    return pl.pallas_call(
        _final_body,
        grid=grid,
        in_specs=[
            pl.BlockSpec((ROW_BLK, NCLS), lambda i: (i, 0)),
            pl.BlockSpec((ROW_BLK, 1), lambda i: (i, 0)),
            pl.BlockSpec((1, NCLS), lambda i: (0, 0)),
        ],
        out_specs=pl.BlockSpec((ROW_BLK, NCLS), lambda i: (i, 0)),
        out_shape=jax.ShapeDtypeStruct((N, NCLS), jnp.float32),
    )(out2, denom2, b2.reshape(1, NCLS))


def _edge_stage(h, a_src, a_dst, src, dst, heads):
    n = h.shape[0]
    alpha = jax.nn.leaky_relu(a_src[src] + a_dst[dst], negative_slope=0.2)
    ex = jnp.exp(alpha)
    denom = jax.ops.segment_sum(ex, dst, num_segments=n)
    msg = h.reshape(n, heads, -1)[src] * ex[:, :, None]
    out = jax.ops.segment_sum(msg, dst, num_segments=n)
    return out, denom


def _build_a1(att_src1, att_dst1):
    eye = jnp.eye(HEADS, dtype=jnp.float32)
    a_s = (att_src1[:, :, None] * eye[:, None, :]).reshape(HEADS * HID, HEADS)
    a_d = (att_dst1[:, :, None] * eye[:, None, :]).reshape(HEADS * HID, HEADS)
    return jnp.concatenate([a_s, a_d], axis=1)


def kernel(x, edge_index, W1, att_src1, att_dst1, b1, W2, att_src2, att_dst2, b2):
    loop = jnp.arange(N, dtype=edge_index.dtype)
    src = jnp.concatenate([edge_index[0], loop])
    dst = jnp.concatenate([edge_index[1], loop])

    A1 = _build_a1(att_src1, att_dst1)
    h, ab = _layer1_matmul(x, W1, A1)
    a_src1v, a_dst1v = ab[:, :HEADS], ab[:, HEADS:]

    out1, denom1 = _edge_stage(h, a_src1v, a_dst1v, src, dst, HEADS)
    act = jax.nn.elu(
        (out1 / (denom1[:, :, None] + 1e-16)).reshape(N, HEADS * HID) + b1
    )

    A2 = jnp.concatenate([att_src2.T, att_dst2.T], axis=1)
    h2, ab2 = _layer2_matmul(act, W2, A2)
    out2, denom2 = _edge_stage(h2, ab2[:, :1], ab2[:, 1:], src, dst, 1)
    out2 = out2.reshape(N, NCLS)

    return _final_logsoftmax(out2, denom2, b2)


# speedup vs baseline: 4.8755x; 4.3476x over previous
import functools
import jax
import jax.numpy as jnp
from jax import lax
from jax.experimental import pallas as pl
from jax.experimental.pallas import tpu as pltpu
from jax.experimental.pallas import tpu_sc as plsc

N = 10000
E = 160000
F_IN = 256
HID = 256
HEADS = 8
NCLS = 40

NPAD = 10240
ETOT = E + N
CH = 2048
EPAD = 172032
D1 = HEADS * HID
COLB = 512
ROW_BLK = 400

_MESH = plsc.VectorSubcoreMesh(core_axis_name="c", subcore_axis_name="s")
_SC_PARAMS = pltpu.CompilerParams(needs_layout_passes=False)


def _mm1_body(xT_ref, w_ref, a_ref, hT_ref, aT_ref):
    h = jnp.dot(w_ref[...], xT_ref[...], preferred_element_type=jnp.float32)
    hT_ref[...] = h
    aT_ref[...] = jnp.dot(a_ref[...], h, preferred_element_type=jnp.float32)


def _layer1_matmul(xT, W1T, A1T):
    return pl.pallas_call(
        _mm1_body,
        grid=(NPAD // COLB,),
        in_specs=[
            pl.BlockSpec((F_IN, COLB), lambda j: (0, j)),
            pl.BlockSpec((D1, F_IN), lambda j: (0, 0)),
            pl.BlockSpec((16, D1), lambda j: (0, 0)),
        ],
        out_specs=[
            pl.BlockSpec((D1, COLB), lambda j: (0, j)),
            pl.BlockSpec((16, COLB), lambda j: (0, j)),
        ],
        out_shape=[
            jax.ShapeDtypeStruct((D1, NPAD), jnp.float32),
            jax.ShapeDtypeStruct((16, NPAD), jnp.float32),
        ],
    )(xT, W1T, A1T)


def _k1_body(aT_hbm, src_hbm, dst_hbm, exT_hbm, denomP_hbm,
             as_tab, ad_tab, dn_tab, sbuf, dbuf, exbuf):
    wid = lax.axis_index("s") * 2 + lax.axis_index("c")
    head = wid // 4
    sl = wid % 4
    pltpu.sync_copy(aT_hbm.at[head], as_tab)
    pltpu.sync_copy(aT_hbm.at[8 + head], ad_tab)

    def _zero(j, _):
        dn_tab[pl.ds(j * 16, 16)] = jnp.zeros((16,), jnp.float32)
        return 0

    lax.fori_loop(0, NPAD // 16, _zero, 0)

    esl = EPAD // 4
    e_base = sl * esl

    def _chunk(ch, _):
        off = e_base + ch * CH
        pltpu.sync_copy(src_hbm.at[pl.ds(off, CH)], sbuf)
        pltpu.sync_copy(dst_hbm.at[pl.ds(off, CH)], dbuf)

        def _iter(i, _):
            svec = sbuf[pl.ds(i * 16, 16)]
            dvec = dbuf[pl.ds(i * 16, 16)]
            z = (plsc.load_gather(as_tab, [svec])
                 + plsc.load_gather(ad_tab, [dvec]))
            ex = jnp.exp(jnp.maximum(z, 0.2 * z))
            exbuf[pl.ds(i * 16, 16)] = ex
            plsc.addupdate_scatter(dn_tab, [dvec], ex)
            return 0

        lax.fori_loop(0, CH // 16, _iter, 0)
        pltpu.sync_copy(exbuf, exT_hbm.at[head, pl.ds(off, CH)])
        return 0

    lax.fori_loop(0, esl // CH, _chunk, 0)
    pltpu.sync_copy(dn_tab, denomP_hbm.at[wid])


_k1 = functools.partial(
    pl.kernel,
    out_type=[
        jax.ShapeDtypeStruct((HEADS, EPAD), jnp.float32),
        jax.ShapeDtypeStruct((32, NPAD), jnp.float32),
    ],
    mesh=_MESH,
    compiler_params=_SC_PARAMS,
    scratch_types=[
        pltpu.VMEM((NPAD,), jnp.float32),
        pltpu.VMEM((NPAD,), jnp.float32),
        pltpu.VMEM((NPAD,), jnp.float32),
        pltpu.VMEM((CH,), jnp.int32),
        pltpu.VMEM((CH,), jnp.int32),
        pltpu.VMEM((CH,), jnp.float32),
    ],
)(_k1_body)


def _k2_body(hT_hbm, exT_hbm, src_hbm, dst_hbm, out1T_hbm,
             h0, h1, h2, h3, a0, a1, a2, a3, sbuf, dbuf, ebuf, sem):
    wid = lax.axis_index("s") * 2 + lax.axis_index("c")
    head = wid // 4
    col0 = head * HID + (wid % 4) * 64
    htabs = (h0, h1, h2, h3)
    atabs = (a0, a1, a2, a3)
    nch = EPAD // CH

    def _start(ch):
        off = ch * CH
        so = lax.rem(ch, 2) * CH
        pltpu.async_copy(src_hbm.at[pl.ds(off, CH)], sbuf.at[pl.ds(so, CH)], sem)
        pltpu.async_copy(dst_hbm.at[pl.ds(off, CH)], dbuf.at[pl.ds(so, CH)], sem)
        pltpu.async_copy(exT_hbm.at[head, pl.ds(off, CH)], ebuf.at[pl.ds(so, CH)], sem)

    def _wait(ch):
        off = ch * CH
        so = lax.rem(ch, 2) * CH
        pltpu.make_async_copy(src_hbm.at[pl.ds(off, CH)], sbuf.at[pl.ds(so, CH)], sem).wait()
        pltpu.make_async_copy(dst_hbm.at[pl.ds(off, CH)], dbuf.at[pl.ds(so, CH)], sem).wait()
        pltpu.make_async_copy(exT_hbm.at[head, pl.ds(off, CH)], ebuf.at[pl.ds(so, CH)], sem).wait()

    def _batch(b, _):
        row0 = col0 + b * 4
        for c in range(4):
            pltpu.sync_copy(hT_hbm.at[row0 + c], htabs[c])

        def _zero(j, _):
            z = jnp.zeros((16,), jnp.float32)
            for c in range(4):
                atabs[c][pl.ds(j * 16, 16)] = z
            return 0

        lax.fori_loop(0, NPAD // 16, _zero, 0)

        _start(0)

        def _chunk(ch, _):
            _wait(ch)

            @pl.when(ch + 1 < nch)
            def _():
                _start(ch + 1)

            so = lax.rem(ch, 2) * CH

            def _iter(i, _):
                base = so + i * 16
                svec = sbuf[pl.ds(base, 16)]
                dvec = dbuf[pl.ds(base, 16)]
                evec = ebuf[pl.ds(base, 16)]
                for c in range(4):
                    v = plsc.load_gather(htabs[c], [svec])
                    plsc.addupdate_scatter(atabs[c], [dvec], v * evec)
                return 0

            lax.fori_loop(0, CH // 16, _iter, 0)
            return 0

        lax.fori_loop(0, nch, _chunk, 0)
        for c in range(4):
            pltpu.sync_copy(atabs[c], out1T_hbm.at[row0 + c])
        return 0

    lax.fori_loop(0, 16, _batch, 0)


_k2 = functools.partial(
    pl.kernel,
    out_type=jax.ShapeDtypeStruct((D1, NPAD), jnp.float32),
    mesh=_MESH,
    compiler_params=_SC_PARAMS,
    scratch_types=(
        [pltpu.VMEM((NPAD,), jnp.float32)] * 8
        + [pltpu.VMEM((2 * CH,), jnp.int32),
           pltpu.VMEM((2 * CH,), jnp.int32),
           pltpu.VMEM((2 * CH,), jnp.float32),
           pltpu.SemaphoreType.DMA]
    ),
)(_k2_body)


def _norm_body(o_ref, d_ref, b_ref, act_ref):
    dsum = jnp.sum(d_ref[0], axis=0, keepdims=True)
    z = o_ref[...] / (dsum + 1e-16) + b_ref[...]
    act_ref[...] = jnp.where(z > 0, z, jnp.exp(jnp.minimum(z, 0.0)) - 1.0)


def _normalize(out1T, denomP, b1):
    return pl.pallas_call(
        _norm_body,
        grid=(HEADS, NPAD // COLB),
        in_specs=[
            pl.BlockSpec((HID, COLB), lambda k, j: (k, j)),
            pl.BlockSpec((1, 4, COLB), lambda k, j: (k, 0, j)),
            pl.BlockSpec((HID, 1), lambda k, j: (k, 0)),
        ],
        out_specs=pl.BlockSpec((HID, COLB), lambda k, j: (k, j)),
        out_shape=jax.ShapeDtypeStruct((D1, NPAD), jnp.float32),
    )(out1T, denomP.reshape(HEADS, 4, NPAD), b1.reshape(D1, 1))


def _mm2_body(act_ref, w_ref, a_ref, h2_ref, ab_ref):
    h2 = jnp.dot(w_ref[...], act_ref[...], preferred_element_type=jnp.float32)
    h2_ref[...] = h2
    ab_ref[...] = jnp.dot(a_ref[...], h2, preferred_element_type=jnp.float32)


def _layer2_matmul(actT, W2Tp, att2p):
    return pl.pallas_call(
        _mm2_body,
        grid=(NPAD // COLB,),
        in_specs=[
            pl.BlockSpec((D1, COLB), lambda j: (0, j)),
            pl.BlockSpec((48, D1), lambda j: (0, 0)),
            pl.BlockSpec((2, 48), lambda j: (0, 0)),
        ],
        out_specs=[
            pl.BlockSpec((48, COLB), lambda j: (0, j)),
            pl.BlockSpec((2, COLB), lambda j: (0, j)),
        ],
        out_shape=[
            jax.ShapeDtypeStruct((48, NPAD), jnp.float32),
            jax.ShapeDtypeStruct((2, NPAD), jnp.float32),
        ],
    )(actT, W2Tp, att2p)


def _k3_body(h2T_hbm, ab2T_hbm, src_hbm, dst_hbm, out2T_hbm, den2_hbm,
             as_tab, ad_tab, t0, t1, a0, a1, dn, sbuf, dbuf):
    wid = lax.axis_index("s") * 2 + lax.axis_index("c")

    @pl.when(wid <= 24)
    def _():
        pltpu.sync_copy(ab2T_hbm.at[0], as_tab)
        pltpu.sync_copy(ab2T_hbm.at[1], ad_tab)
        c0 = wid * 2

        @pl.when(wid < 24)
        def _():
            pltpu.sync_copy(h2T_hbm.at[c0], t0)
            pltpu.sync_copy(h2T_hbm.at[c0 + 1], t1)

        def _zero(j, _):
            z = jnp.zeros((16,), jnp.float32)
            a0[pl.ds(j * 16, 16)] = z
            a1[pl.ds(j * 16, 16)] = z
            dn[pl.ds(j * 16, 16)] = z
            return 0

        lax.fori_loop(0, NPAD // 16, _zero, 0)

        def _chunk(ch, _):
            off = ch * CH
            pltpu.sync_copy(src_hbm.at[pl.ds(off, CH)], sbuf)
            pltpu.sync_copy(dst_hbm.at[pl.ds(off, CH)], dbuf)

            def _iter(i, _):
                svec = sbuf[pl.ds(i * 16, 16)]
                dvec = dbuf[pl.ds(i * 16, 16)]
                z = (plsc.load_gather(as_tab, [svec])
                     + plsc.load_gather(ad_tab, [dvec]))
                ex = jnp.exp(jnp.maximum(z, 0.2 * z))

                @pl.when(wid < 24)
                def _():
                    for tt, aa in ((t0, a0), (t1, a1)):
                        v = plsc.load_gather(tt, [svec])
                        plsc.addupdate_scatter(aa, [dvec], v * ex)

                @pl.when(wid == 24)
                def _():
                    plsc.addupdate_scatter(dn, [dvec], ex)

                return 0

            lax.fori_loop(0, CH // 16, _iter, 0)
            return 0

        lax.fori_loop(0, EPAD // CH, _chunk, 0)

        @pl.when(wid < 24)
        def _():
            pltpu.sync_copy(a0, out2T_hbm.at[c0])
            pltpu.sync_copy(a1, out2T_hbm.at[c0 + 1])

        @pl.when(wid == 24)
        def _():
            pltpu.sync_copy(dn, den2_hbm.at[0])


_k3 = functools.partial(
    pl.kernel,
    out_type=[
        jax.ShapeDtypeStruct((48, NPAD), jnp.float32),
        jax.ShapeDtypeStruct((1, NPAD), jnp.float32),
    ],
    mesh=_MESH,
    compiler_params=_SC_PARAMS,
    scratch_types=(
        [pltpu.VMEM((NPAD,), jnp.float32)] * 7
        + [pltpu.VMEM((CH,), jnp.int32), pltpu.VMEM((CH,), jnp.int32)]
    ),
)(_k3_body)


def _final_body(o_ref, d_ref, b_ref, out_ref):
    z = o_ref[...] / (d_ref[...] + 1e-16) + b_ref[...]
    m = jnp.max(z, axis=1, keepdims=True)
    s = jnp.sum(jnp.exp(z - m), axis=1, keepdims=True)
    out_ref[...] = z - m - jnp.log(s)


def _final_logsoftmax(out2, denom2, b2):
    return pl.pallas_call(
        _final_body,
        grid=(N // ROW_BLK,),
        in_specs=[
            pl.BlockSpec((ROW_BLK, NCLS), lambda i: (i, 0)),
            pl.BlockSpec((ROW_BLK, 1), lambda i: (i, 0)),
            pl.BlockSpec((1, NCLS), lambda i: (0, 0)),
        ],
        out_specs=pl.BlockSpec((ROW_BLK, NCLS), lambda i: (i, 0)),
        out_shape=jax.ShapeDtypeStruct((N, NCLS), jnp.float32),
    )(out2, denom2, b2.reshape(1, NCLS))


def _build_a1(att_src1, att_dst1):
    eye = jnp.eye(HEADS, dtype=jnp.float32)
    a_s = (att_src1[:, :, None] * eye[:, None, :]).reshape(D1, HEADS)
    a_d = (att_dst1[:, :, None] * eye[:, None, :]).reshape(D1, HEADS)
    return jnp.concatenate([a_s, a_d], axis=1)


def kernel(x, edge_index, W1, att_src1, att_dst1, b1, W2, att_src2, att_dst2, b2):
    loop = jnp.arange(N, dtype=jnp.int32)
    padv = jnp.full((EPAD - ETOT,), N, dtype=jnp.int32)
    src = jnp.concatenate([edge_index[0].astype(jnp.int32), loop, padv])
    dst = jnp.concatenate([edge_index[1].astype(jnp.int32), loop, padv])

    xT = jnp.pad(x.T, ((0, 0), (0, NPAD - N)))
    W1T = W1.T
    A1T = _build_a1(att_src1, att_dst1).T

    hT, aT = _layer1_matmul(xT, W1T, A1T)
    exT, denomP = _k1(aT, src, dst)
    out1T = _k2(hT, exT, src, dst)
    actT = _normalize(out1T, denomP, b1)

    W2Tp = jnp.pad(W2.T, ((0, 48 - NCLS), (0, 0)))
    att2p = jnp.pad(jnp.concatenate([att_src2, att_dst2], axis=0),
                    ((0, 0), (0, 48 - NCLS)))
    h2T, ab2T = _layer2_matmul(actT, W2Tp, att2p)
    out2T, den2T = _k3(h2T, ab2T, src, dst)

    out2 = out2T[:NCLS, :N].T
    denom2 = den2T[0, :N].reshape(N, 1)
    return _final_logsoftmax(out2, denom2, b2)


# speedup vs baseline: 5.0554x; 1.0369x over previous
import functools
import jax
import jax.numpy as jnp
from jax import lax
from jax.experimental import pallas as pl
from jax.experimental.pallas import tpu as pltpu
from jax.experimental.pallas import tpu_sc as plsc

N = 10000
E = 160000
F_IN = 256
HID = 256
HEADS = 8
NCLS = 40

NPAD = 10240
ETOT = E + N
CH = 2048
EPAD = 172032
D1 = HEADS * HID
COLB = 512
ROW_BLK = 400

_MESH = plsc.VectorSubcoreMesh(core_axis_name="c", subcore_axis_name="s")
_SC_PARAMS = pltpu.CompilerParams(needs_layout_passes=False)


def _mm1_body(xT_ref, w_ref, a_ref, hT_ref, aT_ref):
    h = jnp.dot(w_ref[...], xT_ref[...], preferred_element_type=jnp.float32)
    hT_ref[...] = h
    aT_ref[...] = jnp.dot(a_ref[...], h, preferred_element_type=jnp.float32)


def _layer1_matmul(xT, W1T, A1T):
    return pl.pallas_call(
        _mm1_body,
        grid=(NPAD // COLB,),
        in_specs=[
            pl.BlockSpec((F_IN, COLB), lambda j: (0, j)),
            pl.BlockSpec((D1, F_IN), lambda j: (0, 0)),
            pl.BlockSpec((16, D1), lambda j: (0, 0)),
        ],
        out_specs=[
            pl.BlockSpec((D1, COLB), lambda j: (0, j)),
            pl.BlockSpec((16, COLB), lambda j: (0, j)),
        ],
        out_shape=[
            jax.ShapeDtypeStruct((D1, NPAD), jnp.float32),
            jax.ShapeDtypeStruct((16, NPAD), jnp.float32),
        ],
    )(xT, W1T, A1T)


def _k1_body(aT_hbm, src_hbm, dst_hbm, exT_hbm, denomP_hbm,
             as_tab, ad_tab, dn_tab, sbuf, dbuf, exbuf):
    wid = lax.axis_index("s") * 2 + lax.axis_index("c")
    head = wid // 4
    sl = wid % 4
    pltpu.sync_copy(aT_hbm.at[head], as_tab)
    pltpu.sync_copy(aT_hbm.at[8 + head], ad_tab)

    def _zero(j, _):
        dn_tab[pl.ds(j * 16, 16)] = jnp.zeros((16,), jnp.float32)
        return 0

    lax.fori_loop(0, NPAD // 16, _zero, 0)

    esl = EPAD // 4
    e_base = sl * esl

    def _chunk(ch, _):
        off = e_base + ch * CH
        pltpu.sync_copy(src_hbm.at[pl.ds(off, CH)], sbuf)
        pltpu.sync_copy(dst_hbm.at[pl.ds(off, CH)], dbuf)

        def _iter(i, _):
            svec = sbuf[pl.ds(i * 16, 16)]
            dvec = dbuf[pl.ds(i * 16, 16)]
            z = (plsc.load_gather(as_tab, [svec])
                 + plsc.load_gather(ad_tab, [dvec]))
            ex = jnp.exp(jnp.maximum(z, 0.2 * z))
            exbuf[pl.ds(i * 16, 16)] = ex
            plsc.addupdate_scatter(dn_tab, [dvec], ex)
            return 0

        lax.fori_loop(0, CH // 16, _iter, 0, unroll=8)
        pltpu.sync_copy(exbuf, exT_hbm.at[head, pl.ds(off, CH)])
        return 0

    lax.fori_loop(0, esl // CH, _chunk, 0)
    pltpu.sync_copy(dn_tab, denomP_hbm.at[wid])


_k1 = functools.partial(
    pl.kernel,
    out_type=[
        jax.ShapeDtypeStruct((HEADS, EPAD), jnp.float32),
        jax.ShapeDtypeStruct((32, NPAD), jnp.float32),
    ],
    mesh=_MESH,
    compiler_params=_SC_PARAMS,
    scratch_types=[
        pltpu.VMEM((NPAD,), jnp.float32),
        pltpu.VMEM((NPAD,), jnp.float32),
        pltpu.VMEM((NPAD,), jnp.float32),
        pltpu.VMEM((CH,), jnp.int32),
        pltpu.VMEM((CH,), jnp.int32),
        pltpu.VMEM((CH,), jnp.float32),
    ],
)(_k1_body)


def _k2_body(hT_hbm, exT_hbm, src_hbm, dst_hbm, out1T_hbm,
             h0, h1, h2, h3, a0, a1, a2, a3, sbuf, dbuf, ebuf, sem):
    wid = lax.axis_index("s") * 2 + lax.axis_index("c")
    head = wid // 4
    col0 = head * HID + (wid % 4) * 64
    htabs = (h0, h1, h2, h3)
    atabs = (a0, a1, a2, a3)
    nch = EPAD // CH

    def _start(ch):
        off = ch * CH
        so = lax.rem(ch, 2) * CH
        pltpu.async_copy(src_hbm.at[pl.ds(off, CH)], sbuf.at[pl.ds(so, CH)], sem)
        pltpu.async_copy(dst_hbm.at[pl.ds(off, CH)], dbuf.at[pl.ds(so, CH)], sem)
        pltpu.async_copy(exT_hbm.at[head, pl.ds(off, CH)], ebuf.at[pl.ds(so, CH)], sem)

    def _wait(ch):
        off = ch * CH
        so = lax.rem(ch, 2) * CH
        pltpu.make_async_copy(src_hbm.at[pl.ds(off, CH)], sbuf.at[pl.ds(so, CH)], sem).wait()
        pltpu.make_async_copy(dst_hbm.at[pl.ds(off, CH)], dbuf.at[pl.ds(so, CH)], sem).wait()
        pltpu.make_async_copy(exT_hbm.at[head, pl.ds(off, CH)], ebuf.at[pl.ds(so, CH)], sem).wait()

    def _batch(b, _):
        row0 = col0 + b * 4
        for c in range(4):
            pltpu.sync_copy(hT_hbm.at[row0 + c], htabs[c])

        def _zero(j, _):
            z = jnp.zeros((16,), jnp.float32)
            for c in range(4):
                atabs[c][pl.ds(j * 16, 16)] = z
            return 0

        lax.fori_loop(0, NPAD // 16, _zero, 0, unroll=8)

        _start(0)

        def _chunk(ch, _):
            _wait(ch)

            @pl.when(ch + 1 < nch)
            def _():
                _start(ch + 1)

            so = lax.rem(ch, 2) * CH

            def _iter(i, _):
                base = so + i * 16
                svec = sbuf[pl.ds(base, 16)]
                dvec = dbuf[pl.ds(base, 16)]
                evec = ebuf[pl.ds(base, 16)]
                for c in range(4):
                    v = plsc.load_gather(htabs[c], [svec])
                    plsc.addupdate_scatter(atabs[c], [dvec], v * evec)
                return 0

            lax.fori_loop(0, CH // 16, _iter, 0, unroll=8)
            return 0

        lax.fori_loop(0, nch, _chunk, 0)
        for c in range(4):
            pltpu.sync_copy(atabs[c], out1T_hbm.at[row0 + c])
        return 0

    lax.fori_loop(0, 16, _batch, 0)


_k2 = functools.partial(
    pl.kernel,
    out_type=jax.ShapeDtypeStruct((D1, NPAD), jnp.float32),
    mesh=_MESH,
    compiler_params=_SC_PARAMS,
    scratch_types=(
        [pltpu.VMEM((NPAD,), jnp.float32)] * 8
        + [pltpu.VMEM((2 * CH,), jnp.int32),
           pltpu.VMEM((2 * CH,), jnp.int32),
           pltpu.VMEM((2 * CH,), jnp.float32),
           pltpu.SemaphoreType.DMA]
    ),
)(_k2_body)


def _norm_body(o_ref, d_ref, b_ref, act_ref):
    dsum = jnp.sum(d_ref[0], axis=0, keepdims=True)
    z = o_ref[...] / (dsum + 1e-16) + b_ref[...]
    act_ref[...] = jnp.where(z > 0, z, jnp.exp(jnp.minimum(z, 0.0)) - 1.0)


def _normalize(out1T, denomP, b1):
    return pl.pallas_call(
        _norm_body,
        grid=(HEADS, NPAD // COLB),
        in_specs=[
            pl.BlockSpec((HID, COLB), lambda k, j: (k, j)),
            pl.BlockSpec((1, 4, COLB), lambda k, j: (k, 0, j)),
            pl.BlockSpec((HID, 1), lambda k, j: (k, 0)),
        ],
        out_specs=pl.BlockSpec((HID, COLB), lambda k, j: (k, j)),
        out_shape=jax.ShapeDtypeStruct((D1, NPAD), jnp.float32),
    )(out1T, denomP.reshape(HEADS, 4, NPAD), b1.reshape(D1, 1))


def _mm2_body(act_ref, w_ref, a_ref, h2_ref, ab_ref):
    h2 = jnp.dot(w_ref[...], act_ref[...], preferred_element_type=jnp.float32)
    h2_ref[...] = h2
    ab_ref[...] = jnp.dot(a_ref[...], h2, preferred_element_type=jnp.float32)


def _layer2_matmul(actT, W2Tp, att2p):
    return pl.pallas_call(
        _mm2_body,
        grid=(NPAD // COLB,),
        in_specs=[
            pl.BlockSpec((D1, COLB), lambda j: (0, j)),
            pl.BlockSpec((48, D1), lambda j: (0, 0)),
            pl.BlockSpec((2, 48), lambda j: (0, 0)),
        ],
        out_specs=[
            pl.BlockSpec((48, COLB), lambda j: (0, j)),
            pl.BlockSpec((2, COLB), lambda j: (0, j)),
        ],
        out_shape=[
            jax.ShapeDtypeStruct((48, NPAD), jnp.float32),
            jax.ShapeDtypeStruct((2, NPAD), jnp.float32),
        ],
    )(actT, W2Tp, att2p)


def _k3_body(h2T_hbm, ab2T_hbm, src_hbm, dst_hbm, out2T_hbm, den2_hbm,
             as_tab, ad_tab, t0, t1, a0, a1, dn, sbuf, dbuf):
    wid = lax.axis_index("s") * 2 + lax.axis_index("c")

    @pl.when(wid <= 24)
    def _():
        pltpu.sync_copy(ab2T_hbm.at[0], as_tab)
        pltpu.sync_copy(ab2T_hbm.at[1], ad_tab)
        c0 = wid * 2

        @pl.when(wid < 24)
        def _():
            pltpu.sync_copy(h2T_hbm.at[c0], t0)
            pltpu.sync_copy(h2T_hbm.at[c0 + 1], t1)

        def _zero(j, _):
            z = jnp.zeros((16,), jnp.float32)
            a0[pl.ds(j * 16, 16)] = z
            a1[pl.ds(j * 16, 16)] = z
            dn[pl.ds(j * 16, 16)] = z
            return 0

        lax.fori_loop(0, NPAD // 16, _zero, 0)

        def _chunk(ch, _):
            off = ch * CH
            pltpu.sync_copy(src_hbm.at[pl.ds(off, CH)], sbuf)
            pltpu.sync_copy(dst_hbm.at[pl.ds(off, CH)], dbuf)

            def _iter(i, _):
                svec = sbuf[pl.ds(i * 16, 16)]
                dvec = dbuf[pl.ds(i * 16, 16)]
                z = (plsc.load_gather(as_tab, [svec])
                     + plsc.load_gather(ad_tab, [dvec]))
                ex = jnp.exp(jnp.maximum(z, 0.2 * z))

                @pl.when(wid < 24)
                def _():
                    for tt, aa in ((t0, a0), (t1, a1)):
                        v = plsc.load_gather(tt, [svec])
                        plsc.addupdate_scatter(aa, [dvec], v * ex)

                @pl.when(wid == 24)
                def _():
                    plsc.addupdate_scatter(dn, [dvec], ex)

                return 0

            lax.fori_loop(0, CH // 16, _iter, 0, unroll=8)
            return 0

        lax.fori_loop(0, EPAD // CH, _chunk, 0)

        @pl.when(wid < 24)
        def _():
            pltpu.sync_copy(a0, out2T_hbm.at[c0])
            pltpu.sync_copy(a1, out2T_hbm.at[c0 + 1])

        @pl.when(wid == 24)
        def _():
            pltpu.sync_copy(dn, den2_hbm.at[0])


_k3 = functools.partial(
    pl.kernel,
    out_type=[
        jax.ShapeDtypeStruct((48, NPAD), jnp.float32),
        jax.ShapeDtypeStruct((1, NPAD), jnp.float32),
    ],
    mesh=_MESH,
    compiler_params=_SC_PARAMS,
    scratch_types=(
        [pltpu.VMEM((NPAD,), jnp.float32)] * 7
        + [pltpu.VMEM((CH,), jnp.int32), pltpu.VMEM((CH,), jnp.int32)]
    ),
)(_k3_body)


def _final_body(o_ref, d_ref, b_ref, out_ref):
    z = o_ref[...] / (d_ref[...] + 1e-16) + b_ref[...]
    m = jnp.max(z, axis=1, keepdims=True)
    s = jnp.sum(jnp.exp(z - m), axis=1, keepdims=True)
    out_ref[...] = z - m - jnp.log(s)


def _final_logsoftmax(out2, denom2, b2):
    return pl.pallas_call(
        _final_body,
        grid=(N // ROW_BLK,),
        in_specs=[
            pl.BlockSpec((ROW_BLK, NCLS), lambda i: (i, 0)),
            pl.BlockSpec((ROW_BLK, 1), lambda i: (i, 0)),
            pl.BlockSpec((1, NCLS), lambda i: (0, 0)),
        ],
        out_specs=pl.BlockSpec((ROW_BLK, NCLS), lambda i: (i, 0)),
        out_shape=jax.ShapeDtypeStruct((N, NCLS), jnp.float32),
    )(out2, denom2, b2.reshape(1, NCLS))


def _build_a1(att_src1, att_dst1):
    eye = jnp.eye(HEADS, dtype=jnp.float32)
    a_s = (att_src1[:, :, None] * eye[:, None, :]).reshape(D1, HEADS)
    a_d = (att_dst1[:, :, None] * eye[:, None, :]).reshape(D1, HEADS)
    return jnp.concatenate([a_s, a_d], axis=1)


def kernel(x, edge_index, W1, att_src1, att_dst1, b1, W2, att_src2, att_dst2, b2):
    loop = jnp.arange(N, dtype=jnp.int32)
    padv = jnp.full((EPAD - ETOT,), N, dtype=jnp.int32)
    src = jnp.concatenate([edge_index[0].astype(jnp.int32), loop, padv])
    dst = jnp.concatenate([edge_index[1].astype(jnp.int32), loop, padv])

    xT = jnp.pad(x.T, ((0, 0), (0, NPAD - N)))
    W1T = W1.T
    A1T = _build_a1(att_src1, att_dst1).T

    hT, aT = _layer1_matmul(xT, W1T, A1T)
    exT, denomP = _k1(aT, src, dst)
    out1T = _k2(hT, exT, src, dst)
    actT = _normalize(out1T, denomP, b1)

    W2Tp = jnp.pad(W2.T, ((0, 48 - NCLS), (0, 0)))
    att2p = jnp.pad(jnp.concatenate([att_src2, att_dst2], axis=0),
                    ((0, 0), (0, 48 - NCLS)))
    h2T, ab2T = _layer2_matmul(actT, W2Tp, att2p)
    out2T, den2T = _k3(h2T, ab2T, src, dst)

    out2 = out2T[:NCLS, :N].T
    denom2 = den2T[0, :N].reshape(N, 1)
    return _final_logsoftmax(out2, denom2, b2)


# speedup vs baseline: 12.1401x; 2.4014x over previous
import functools
import jax
import jax.numpy as jnp
from jax import lax
from jax.experimental import pallas as pl
from jax.experimental.pallas import tpu as pltpu
from jax.experimental.pallas import tpu_sc as plsc

N = 10000
E = 160000
F_IN = 256
HID = 256
HEADS = 8
NCLS = 40

NPAD = 10240
ETOT = E + N
CH = 2048
EPAD = 172032
D1 = HEADS * HID
COLB = 512
ROW_BLK = 400

_MESH = plsc.VectorSubcoreMesh(core_axis_name="c", subcore_axis_name="s")
_SC_PARAMS = pltpu.CompilerParams(needs_layout_passes=False)


def _mm1_body(xT_ref, w_ref, a_ref, hT_ref, aT_ref):
    h = jnp.dot(w_ref[...], xT_ref[...], preferred_element_type=jnp.float32)
    hT_ref[...] = h
    aT_ref[...] = jnp.dot(a_ref[...], h, preferred_element_type=jnp.float32)


def _layer1_matmul(xT, W1T, A1T):
    return pl.pallas_call(
        _mm1_body,
        grid=(NPAD // COLB,),
        in_specs=[
            pl.BlockSpec((F_IN, COLB), lambda j: (0, j)),
            pl.BlockSpec((D1, F_IN), lambda j: (0, 0)),
            pl.BlockSpec((16, D1), lambda j: (0, 0)),
        ],
        out_specs=[
            pl.BlockSpec((D1, COLB), lambda j: (0, j)),
            pl.BlockSpec((16, COLB), lambda j: (0, j)),
        ],
        out_shape=[
            jax.ShapeDtypeStruct((D1, NPAD), jnp.float32),
            jax.ShapeDtypeStruct((16, NPAD), jnp.float32),
        ],
    )(xT, W1T, A1T)


def _k1_body(aT_hbm, src_hbm, dst_hbm, exT_hbm, denomP_hbm,
             as_tab, ad_tab, dn_tab, sbuf, dbuf, exbuf):
    wid = lax.axis_index("s") * 2 + lax.axis_index("c")
    head = wid // 4
    sl = wid % 4
    pltpu.sync_copy(aT_hbm.at[head], as_tab)
    pltpu.sync_copy(aT_hbm.at[8 + head], ad_tab)

    def _zero(j, _):
        dn_tab[pl.ds(j * 16, 16)] = jnp.zeros((16,), jnp.float32)
        return 0

    lax.fori_loop(0, NPAD // 16, _zero, 0)

    esl = EPAD // 4
    e_base = sl * esl

    def _chunk(ch, _):
        off = e_base + ch * CH
        pltpu.sync_copy(src_hbm.at[pl.ds(off, CH)], sbuf)
        pltpu.sync_copy(dst_hbm.at[pl.ds(off, CH)], dbuf)

        @plsc.parallel_loop(0, CH // 16, unroll=8)
        def _iter(i):
            svec = sbuf[pl.ds(i * 16, 16)]
            dvec = dbuf[pl.ds(i * 16, 16)]
            z = (plsc.load_gather(as_tab, [svec])
                 + plsc.load_gather(ad_tab, [dvec]))
            ex = jnp.exp(jnp.maximum(z, 0.2 * z))
            exbuf[pl.ds(i * 16, 16)] = ex
            plsc.addupdate_scatter(dn_tab, [dvec], ex)
        pltpu.sync_copy(exbuf, exT_hbm.at[head, pl.ds(off, CH)])
        return 0

    lax.fori_loop(0, esl // CH, _chunk, 0)
    pltpu.sync_copy(dn_tab, denomP_hbm.at[wid])


_k1 = functools.partial(
    pl.kernel,
    out_type=[
        jax.ShapeDtypeStruct((HEADS, EPAD), jnp.float32),
        jax.ShapeDtypeStruct((32, NPAD), jnp.float32),
    ],
    mesh=_MESH,
    compiler_params=_SC_PARAMS,
    scratch_types=[
        pltpu.VMEM((NPAD,), jnp.float32),
        pltpu.VMEM((NPAD,), jnp.float32),
        pltpu.VMEM((NPAD,), jnp.float32),
        pltpu.VMEM((CH,), jnp.int32),
        pltpu.VMEM((CH,), jnp.int32),
        pltpu.VMEM((CH,), jnp.float32),
    ],
)(_k1_body)


def _k2_body(hT_hbm, exT_hbm, src_hbm, dst_hbm, out1T_hbm,
             h0, h1, h2, h3, a0, a1, a2, a3, sbuf, dbuf, ebuf, sem):
    wid = lax.axis_index("s") * 2 + lax.axis_index("c")
    head = wid // 4
    col0 = head * HID + (wid % 4) * 64
    htabs = (h0, h1, h2, h3)
    atabs = (a0, a1, a2, a3)
    nch = EPAD // CH

    def _start(ch):
        off = ch * CH
        so = lax.rem(ch, 2) * CH
        pltpu.async_copy(src_hbm.at[pl.ds(off, CH)], sbuf.at[pl.ds(so, CH)], sem)
        pltpu.async_copy(dst_hbm.at[pl.ds(off, CH)], dbuf.at[pl.ds(so, CH)], sem)
        pltpu.async_copy(exT_hbm.at[head, pl.ds(off, CH)], ebuf.at[pl.ds(so, CH)], sem)

    def _wait(ch):
        off = ch * CH
        so = lax.rem(ch, 2) * CH
        pltpu.make_async_copy(src_hbm.at[pl.ds(off, CH)], sbuf.at[pl.ds(so, CH)], sem).wait()
        pltpu.make_async_copy(dst_hbm.at[pl.ds(off, CH)], dbuf.at[pl.ds(so, CH)], sem).wait()
        pltpu.make_async_copy(exT_hbm.at[head, pl.ds(off, CH)], ebuf.at[pl.ds(so, CH)], sem).wait()

    def _batch(b, _):
        row0 = col0 + b * 4
        for c in range(4):
            pltpu.sync_copy(hT_hbm.at[row0 + c], htabs[c])

        def _zero(j, _):
            z = jnp.zeros((16,), jnp.float32)
            for c in range(4):
                atabs[c][pl.ds(j * 16, 16)] = z
            return 0

        lax.fori_loop(0, NPAD // 16, _zero, 0, unroll=8)

        _start(0)

        def _chunk(ch, _):
            _wait(ch)

            @pl.when(ch + 1 < nch)
            def _():
                _start(ch + 1)

            so = lax.rem(ch, 2) * CH

            @plsc.parallel_loop(0, CH // 16, unroll=8)
            def _iter(i):
                base = so + i * 16
                svec = sbuf[pl.ds(base, 16)]
                dvec = dbuf[pl.ds(base, 16)]
                evec = ebuf[pl.ds(base, 16)]
                for c in range(4):
                    v = plsc.load_gather(htabs[c], [svec])
                    plsc.addupdate_scatter(atabs[c], [dvec], v * evec)
            return 0

        lax.fori_loop(0, nch, _chunk, 0)
        for c in range(4):
            pltpu.sync_copy(atabs[c], out1T_hbm.at[row0 + c])
        return 0

    lax.fori_loop(0, 16, _batch, 0)


_k2 = functools.partial(
    pl.kernel,
    out_type=jax.ShapeDtypeStruct((D1, NPAD), jnp.float32),
    mesh=_MESH,
    compiler_params=_SC_PARAMS,
    scratch_types=(
        [pltpu.VMEM((NPAD,), jnp.float32)] * 8
        + [pltpu.VMEM((2 * CH,), jnp.int32),
           pltpu.VMEM((2 * CH,), jnp.int32),
           pltpu.VMEM((2 * CH,), jnp.float32),
           pltpu.SemaphoreType.DMA]
    ),
)(_k2_body)


def _norm_body(o_ref, d_ref, b_ref, act_ref):
    dsum = jnp.sum(d_ref[0], axis=0, keepdims=True)
    z = o_ref[...] / (dsum + 1e-16) + b_ref[...]
    act_ref[...] = jnp.where(z > 0, z, jnp.exp(jnp.minimum(z, 0.0)) - 1.0)


def _normalize(out1T, denomP, b1):
    return pl.pallas_call(
        _norm_body,
        grid=(HEADS, NPAD // COLB),
        in_specs=[
            pl.BlockSpec((HID, COLB), lambda k, j: (k, j)),
            pl.BlockSpec((1, 4, COLB), lambda k, j: (k, 0, j)),
            pl.BlockSpec((HID, 1), lambda k, j: (k, 0)),
        ],
        out_specs=pl.BlockSpec((HID, COLB), lambda k, j: (k, j)),
        out_shape=jax.ShapeDtypeStruct((D1, NPAD), jnp.float32),
    )(out1T, denomP.reshape(HEADS, 4, NPAD), b1.reshape(D1, 1))


def _mm2_body(act_ref, w_ref, a_ref, h2_ref, ab_ref):
    h2 = jnp.dot(w_ref[...], act_ref[...], preferred_element_type=jnp.float32)
    h2_ref[...] = h2
    ab_ref[...] = jnp.dot(a_ref[...], h2, preferred_element_type=jnp.float32)


def _layer2_matmul(actT, W2Tp, att2p):
    return pl.pallas_call(
        _mm2_body,
        grid=(NPAD // COLB,),
        in_specs=[
            pl.BlockSpec((D1, COLB), lambda j: (0, j)),
            pl.BlockSpec((48, D1), lambda j: (0, 0)),
            pl.BlockSpec((2, 48), lambda j: (0, 0)),
        ],
        out_specs=[
            pl.BlockSpec((48, COLB), lambda j: (0, j)),
            pl.BlockSpec((2, COLB), lambda j: (0, j)),
        ],
        out_shape=[
            jax.ShapeDtypeStruct((48, NPAD), jnp.float32),
            jax.ShapeDtypeStruct((2, NPAD), jnp.float32),
        ],
    )(actT, W2Tp, att2p)


def _k3_body(h2T_hbm, ab2T_hbm, src_hbm, dst_hbm, out2T_hbm, den2_hbm,
             as_tab, ad_tab, t0, t1, a0, a1, dn, sbuf, dbuf):
    wid = lax.axis_index("s") * 2 + lax.axis_index("c")

    @pl.when(wid <= 24)
    def _():
        pltpu.sync_copy(ab2T_hbm.at[0], as_tab)
        pltpu.sync_copy(ab2T_hbm.at[1], ad_tab)
        c0 = wid * 2

        @pl.when(wid < 24)
        def _():
            pltpu.sync_copy(h2T_hbm.at[c0], t0)
            pltpu.sync_copy(h2T_hbm.at[c0 + 1], t1)

        def _zero(j, _):
            z = jnp.zeros((16,), jnp.float32)
            a0[pl.ds(j * 16, 16)] = z
            a1[pl.ds(j * 16, 16)] = z
            dn[pl.ds(j * 16, 16)] = z
            return 0

        lax.fori_loop(0, NPAD // 16, _zero, 0)

        def _chunk(ch, _):
            off = ch * CH
            pltpu.sync_copy(src_hbm.at[pl.ds(off, CH)], sbuf)
            pltpu.sync_copy(dst_hbm.at[pl.ds(off, CH)], dbuf)

            @plsc.parallel_loop(0, CH // 16, unroll=8)
            def _iter(i):
                svec = sbuf[pl.ds(i * 16, 16)]
                dvec = dbuf[pl.ds(i * 16, 16)]
                z = (plsc.load_gather(as_tab, [svec])
                     + plsc.load_gather(ad_tab, [dvec]))
                ex = jnp.exp(jnp.maximum(z, 0.2 * z))

                @pl.when(wid < 24)
                def _():
                    for tt, aa in ((t0, a0), (t1, a1)):
                        v = plsc.load_gather(tt, [svec])
                        plsc.addupdate_scatter(aa, [dvec], v * ex)

                @pl.when(wid == 24)
                def _():
                    plsc.addupdate_scatter(dn, [dvec], ex)
            return 0

        lax.fori_loop(0, EPAD // CH, _chunk, 0)

        @pl.when(wid < 24)
        def _():
            pltpu.sync_copy(a0, out2T_hbm.at[c0])
            pltpu.sync_copy(a1, out2T_hbm.at[c0 + 1])

        @pl.when(wid == 24)
        def _():
            pltpu.sync_copy(dn, den2_hbm.at[0])


_k3 = functools.partial(
    pl.kernel,
    out_type=[
        jax.ShapeDtypeStruct((48, NPAD), jnp.float32),
        jax.ShapeDtypeStruct((1, NPAD), jnp.float32),
    ],
    mesh=_MESH,
    compiler_params=_SC_PARAMS,
    scratch_types=(
        [pltpu.VMEM((NPAD,), jnp.float32)] * 7
        + [pltpu.VMEM((CH,), jnp.int32), pltpu.VMEM((CH,), jnp.int32)]
    ),
)(_k3_body)


def _final_body(o_ref, d_ref, b_ref, out_ref):
    z = o_ref[...] / (d_ref[...] + 1e-16) + b_ref[...]
    m = jnp.max(z, axis=1, keepdims=True)
    s = jnp.sum(jnp.exp(z - m), axis=1, keepdims=True)
    out_ref[...] = z - m - jnp.log(s)


def _final_logsoftmax(out2, denom2, b2):
    return pl.pallas_call(
        _final_body,
        grid=(N // ROW_BLK,),
        in_specs=[
            pl.BlockSpec((ROW_BLK, NCLS), lambda i: (i, 0)),
            pl.BlockSpec((ROW_BLK, 1), lambda i: (i, 0)),
            pl.BlockSpec((1, NCLS), lambda i: (0, 0)),
        ],
        out_specs=pl.BlockSpec((ROW_BLK, NCLS), lambda i: (i, 0)),
        out_shape=jax.ShapeDtypeStruct((N, NCLS), jnp.float32),
    )(out2, denom2, b2.reshape(1, NCLS))


def _build_a1(att_src1, att_dst1):
    eye = jnp.eye(HEADS, dtype=jnp.float32)
    a_s = (att_src1[:, :, None] * eye[:, None, :]).reshape(D1, HEADS)
    a_d = (att_dst1[:, :, None] * eye[:, None, :]).reshape(D1, HEADS)
    return jnp.concatenate([a_s, a_d], axis=1)


def kernel(x, edge_index, W1, att_src1, att_dst1, b1, W2, att_src2, att_dst2, b2):
    loop = jnp.arange(N, dtype=jnp.int32)
    padv = jnp.full((EPAD - ETOT,), N, dtype=jnp.int32)
    src = jnp.concatenate([edge_index[0].astype(jnp.int32), loop, padv])
    dst = jnp.concatenate([edge_index[1].astype(jnp.int32), loop, padv])

    xT = jnp.pad(x.T, ((0, 0), (0, NPAD - N)))
    W1T = W1.T
    A1T = _build_a1(att_src1, att_dst1).T

    hT, aT = _layer1_matmul(xT, W1T, A1T)
    exT, denomP = _k1(aT, src, dst)
    out1T = _k2(hT, exT, src, dst)
    actT = _normalize(out1T, denomP, b1)

    W2Tp = jnp.pad(W2.T, ((0, 48 - NCLS), (0, 0)))
    att2p = jnp.pad(jnp.concatenate([att_src2, att_dst2], axis=0),
                    ((0, 0), (0, 48 - NCLS)))
    h2T, ab2T = _layer2_matmul(actT, W2Tp, att2p)
    out2T, den2T = _k3(h2T, ab2T, src, dst)

    out2 = out2T[:NCLS, :N].T
    denom2 = den2T[0, :N].reshape(N, 1)
    return _final_logsoftmax(out2, denom2, b2)


# speedup vs baseline: 12.8790x; 1.0609x over previous
import functools
import jax
import jax.numpy as jnp
from jax import lax
from jax.experimental import pallas as pl
from jax.experimental.pallas import tpu as pltpu
from jax.experimental.pallas import tpu_sc as plsc

N = 10000
E = 160000
F_IN = 256
HID = 256
HEADS = 8
NCLS = 40

NPAD = 10240
CH = 2048
EPAD = 163840
D1 = HEADS * HID
COLB = 512
PKS = 14

_MESH = plsc.VectorSubcoreMesh(core_axis_name="c", subcore_axis_name="s")
_SC_PARAMS = pltpu.CompilerParams(needs_layout_passes=False)


def _unpack_edges(v):
    svec = v & jnp.int32((1 << PKS) - 1)
    dvec = lax.shift_right_logical(v, jnp.int32(PKS))
    return svec, dvec


def _mm1_body(xT_ref, w_ref, a_ref, hT_ref, aT_ref):
    h = jnp.dot(w_ref[...], xT_ref[...], preferred_element_type=jnp.float32)
    hT_ref[...] = h
    aT_ref[...] = jnp.dot(a_ref[...], h, preferred_element_type=jnp.float32)


def _layer1_matmul(xT, W1T, A1T):
    return pl.pallas_call(
        _mm1_body,
        grid=(NPAD // COLB,),
        in_specs=[
            pl.BlockSpec((F_IN, COLB), lambda j: (0, j)),
            pl.BlockSpec((D1, F_IN), lambda j: (0, 0)),
            pl.BlockSpec((16, D1), lambda j: (0, 0)),
        ],
        out_specs=[
            pl.BlockSpec((D1, COLB), lambda j: (0, j)),
            pl.BlockSpec((16, COLB), lambda j: (0, j)),
        ],
        out_shape=[
            jax.ShapeDtypeStruct((D1, NPAD), jnp.float32),
            jax.ShapeDtypeStruct((16, NPAD), jnp.float32),
        ],
    )(xT, W1T, A1T)


def _k1_body(aT_hbm, pk_hbm, exT_hbm, denomP_hbm,
             as_tab, ad_tab, dn_tab, pkbuf, exbuf):
    wid = lax.axis_index("s") * 2 + lax.axis_index("c")
    head = wid // 4
    sl = wid % 4
    pltpu.sync_copy(aT_hbm.at[head], as_tab)
    pltpu.sync_copy(aT_hbm.at[8 + head], ad_tab)

    @plsc.parallel_loop(0, NPAD // 16, unroll=8)
    def _zero(j):
        dn_tab[pl.ds(j * 16, 16)] = jnp.zeros((16,), jnp.float32)

    esl = EPAD // 4
    e_base = sl * esl

    def _chunk(ch, _):
        off = e_base + ch * CH
        pltpu.sync_copy(pk_hbm.at[pl.ds(off, CH)], pkbuf)

        @plsc.parallel_loop(0, CH // 16, unroll=8)
        def _iter(i):
            svec, dvec = _unpack_edges(pkbuf[pl.ds(i * 16, 16)])
            z = (plsc.load_gather(as_tab, [svec])
                 + plsc.load_gather(ad_tab, [dvec]))
            ex = jnp.exp(jnp.maximum(z, 0.2 * z))
            exbuf[pl.ds(i * 16, 16)] = ex
            plsc.addupdate_scatter(dn_tab, [dvec], ex)

        pltpu.sync_copy(exbuf, exT_hbm.at[head, pl.ds(off, CH)])
        return 0

    lax.fori_loop(0, esl // CH, _chunk, 0)
    pltpu.sync_copy(dn_tab, denomP_hbm.at[wid])


_k1 = functools.partial(
    pl.kernel,
    out_type=[
        jax.ShapeDtypeStruct((HEADS, EPAD), jnp.float32),
        jax.ShapeDtypeStruct((32, NPAD), jnp.float32),
    ],
    mesh=_MESH,
    compiler_params=_SC_PARAMS,
    scratch_types=[
        pltpu.VMEM((NPAD,), jnp.float32),
        pltpu.VMEM((NPAD,), jnp.float32),
        pltpu.VMEM((NPAD,), jnp.float32),
        pltpu.VMEM((CH,), jnp.int32),
        pltpu.VMEM((CH,), jnp.float32),
    ],
)(_k1_body)


def _k2_body(hT_hbm, exT_hbm, pk_hbm, out1T_hbm,
             h0, h1, h2, h3, a0, a1, a2, a3, pkbuf, ebuf, sem):
    wid = lax.axis_index("s") * 2 + lax.axis_index("c")
    head = wid // 4
    col0 = head * HID + (wid % 4) * 64
    htabs = (h0, h1, h2, h3)
    atabs = (a0, a1, a2, a3)
    nch = EPAD // CH

    def _start(ch):
        off = ch * CH
        so = lax.rem(ch, 2) * CH
        pltpu.async_copy(pk_hbm.at[pl.ds(off, CH)], pkbuf.at[pl.ds(so, CH)], sem)
        pltpu.async_copy(exT_hbm.at[head, pl.ds(off, CH)], ebuf.at[pl.ds(so, CH)], sem)

    def _wait(ch):
        off = ch * CH
        so = lax.rem(ch, 2) * CH
        pltpu.make_async_copy(pk_hbm.at[pl.ds(off, CH)], pkbuf.at[pl.ds(so, CH)], sem).wait()
        pltpu.make_async_copy(exT_hbm.at[head, pl.ds(off, CH)], ebuf.at[pl.ds(so, CH)], sem).wait()

    def _batch(b, _):
        row0 = col0 + b * 4
        for c in range(4):
            pltpu.sync_copy(hT_hbm.at[row0 + c], htabs[c])

        @plsc.parallel_loop(0, NPAD // 16, unroll=8)
        def _zero(j):
            z = jnp.zeros((16,), jnp.float32)
            for c in range(4):
                atabs[c][pl.ds(j * 16, 16)] = z

        _start(0)

        def _chunk(ch, _):
            _wait(ch)

            @pl.when(ch + 1 < nch)
            def _():
                _start(ch + 1)

            so = lax.rem(ch, 2) * CH

            @plsc.parallel_loop(0, CH // 16, unroll=8)
            def _iter(i):
                base = so + i * 16
                svec, dvec = _unpack_edges(pkbuf[pl.ds(base, 16)])
                evec = ebuf[pl.ds(base, 16)]
                for c in range(4):
                    v = plsc.load_gather(htabs[c], [svec])
                    plsc.addupdate_scatter(atabs[c], [dvec], v * evec)

            return 0

        lax.fori_loop(0, nch, _chunk, 0)
        for c in range(4):
            pltpu.sync_copy(atabs[c], out1T_hbm.at[row0 + c])
        return 0

    lax.fori_loop(0, 16, _batch, 0)


_k2 = functools.partial(
    pl.kernel,
    out_type=jax.ShapeDtypeStruct((D1, NPAD), jnp.float32),
    mesh=_MESH,
    compiler_params=_SC_PARAMS,
    scratch_types=(
        [pltpu.VMEM((NPAD,), jnp.float32)] * 8
        + [pltpu.VMEM((2 * CH,), jnp.int32),
           pltpu.VMEM((2 * CH,), jnp.float32),
           pltpu.SemaphoreType.DMA]
    ),
)(_k2_body)


def _norm_body(o_ref, d_ref, as_ref, ad_ref, h_ref, b_ref, act_ref):
    za = as_ref[0] + ad_ref[0]
    ex_self = jnp.exp(jnp.maximum(za, 0.2 * za))
    dsum = jnp.sum(d_ref[0], axis=0, keepdims=True) + ex_self
    z = (o_ref[...] + ex_self * h_ref[...]) / (dsum + 1e-16) + b_ref[...]
    act_ref[...] = jnp.where(z > 0, z, jnp.exp(jnp.minimum(z, 0.0)) - 1.0)


def _normalize(out1T, denomP, aT, hT, b1):
    aT3 = aT.reshape(16, 1, NPAD)
    return pl.pallas_call(
        _norm_body,
        grid=(HEADS, NPAD // COLB),
        in_specs=[
            pl.BlockSpec((HID, COLB), lambda k, j: (k, j)),
            pl.BlockSpec((1, 4, COLB), lambda k, j: (k, 0, j)),
            pl.BlockSpec((1, 1, COLB), lambda k, j: (k, 0, j)),
            pl.BlockSpec((1, 1, COLB), lambda k, j: (8 + k, 0, j)),
            pl.BlockSpec((HID, COLB), lambda k, j: (k, j)),
            pl.BlockSpec((HID, 1), lambda k, j: (k, 0)),
        ],
        out_specs=pl.BlockSpec((HID, COLB), lambda k, j: (k, j)),
        out_shape=jax.ShapeDtypeStruct((D1, NPAD), jnp.float32),
    )(out1T, denomP.reshape(HEADS, 4, NPAD), aT3, aT3, hT, b1.reshape(D1, 1))


def _mm2_body(act_ref, w_ref, a_ref, h2_ref, ab_ref):
    h2 = jnp.dot(w_ref[...], act_ref[...], preferred_element_type=jnp.float32)
    h2_ref[...] = h2
    ab_ref[...] = jnp.dot(a_ref[...], h2, preferred_element_type=jnp.float32)


def _layer2_matmul(actT, W2Tp, att2p):
    return pl.pallas_call(
        _mm2_body,
        grid=(NPAD // COLB,),
        in_specs=[
            pl.BlockSpec((D1, COLB), lambda j: (0, j)),
            pl.BlockSpec((48, D1), lambda j: (0, 0)),
            pl.BlockSpec((2, 48), lambda j: (0, 0)),
        ],
        out_specs=[
            pl.BlockSpec((48, COLB), lambda j: (0, j)),
            pl.BlockSpec((2, COLB), lambda j: (0, j)),
        ],
        out_shape=[
            jax.ShapeDtypeStruct((48, NPAD), jnp.float32),
            jax.ShapeDtypeStruct((2, NPAD), jnp.float32),
        ],
    )(actT, W2Tp, att2p)


def _k3_body(h2T_hbm, ab2T_hbm, pk_hbm, out2T_hbm, den2_hbm,
             as_tab, ad_tab, t0, t1, a0, a1, dn, pkbuf):
    wid = lax.axis_index("s") * 2 + lax.axis_index("c")

    @pl.when(wid <= 24)
    def _():
        pltpu.sync_copy(ab2T_hbm.at[0], as_tab)
        pltpu.sync_copy(ab2T_hbm.at[1], ad_tab)
        c0 = wid * 2

        @pl.when(wid < 24)
        def _():
            pltpu.sync_copy(h2T_hbm.at[c0], t0)
            pltpu.sync_copy(h2T_hbm.at[c0 + 1], t1)

        @plsc.parallel_loop(0, NPAD // 16, unroll=8)
        def _zero(j):
            z = jnp.zeros((16,), jnp.float32)
            a0[pl.ds(j * 16, 16)] = z
            a1[pl.ds(j * 16, 16)] = z
            dn[pl.ds(j * 16, 16)] = z

        def _chunk(ch, _):
            off = ch * CH
            pltpu.sync_copy(pk_hbm.at[pl.ds(off, CH)], pkbuf)

            @plsc.parallel_loop(0, CH // 16, unroll=8)
            def _iter(i):
                svec, dvec = _unpack_edges(pkbuf[pl.ds(i * 16, 16)])
                z = (plsc.load_gather(as_tab, [svec])
                     + plsc.load_gather(ad_tab, [dvec]))
                ex = jnp.exp(jnp.maximum(z, 0.2 * z))

                @pl.when(wid < 24)
                def _():
                    for tt, aa in ((t0, a0), (t1, a1)):
                        v = plsc.load_gather(tt, [svec])
                        plsc.addupdate_scatter(aa, [dvec], v * ex)

                @pl.when(wid == 24)
                def _():
                    plsc.addupdate_scatter(dn, [dvec], ex)

            return 0

        lax.fori_loop(0, EPAD // CH, _chunk, 0)

        @pl.when(wid < 24)
        def _():
            pltpu.sync_copy(a0, out2T_hbm.at[c0])
            pltpu.sync_copy(a1, out2T_hbm.at[c0 + 1])

        @pl.when(wid == 24)
        def _():
            pltpu.sync_copy(dn, den2_hbm.at[0])


_k3 = functools.partial(
    pl.kernel,
    out_type=[
        jax.ShapeDtypeStruct((48, NPAD), jnp.float32),
        jax.ShapeDtypeStruct((1, NPAD), jnp.float32),
    ],
    mesh=_MESH,
    compiler_params=_SC_PARAMS,
    scratch_types=(
        [pltpu.VMEM((NPAD,), jnp.float32)] * 7
        + [pltpu.VMEM((CH,), jnp.int32)]
    ),
)(_k3_body)


def _final_body(o_ref, d_ref, as_ref, ad_ref, h_ref, b_ref, out_ref):
    za = as_ref[0] + ad_ref[0]
    ex_self = jnp.exp(jnp.maximum(za, 0.2 * za))
    num = o_ref[...] + ex_self * h_ref[...]
    den = d_ref[...] + ex_self + 1e-16
    z = num / den + b_ref[...]
    rowid = lax.broadcasted_iota(jnp.int32, (48, COLB), 0)
    valid = rowid < NCLS
    zm = jnp.where(valid, z, -1e30)
    m = jnp.max(zm, axis=0, keepdims=True)
    s = jnp.sum(jnp.where(valid, jnp.exp(z - m), 0.0), axis=0, keepdims=True)
    out_ref[...] = z - m - jnp.log(s)


def _final_logsoftmax(out2T, den2T, ab2T, h2T, b2p):
    ab23 = ab2T.reshape(2, 1, NPAD)
    return pl.pallas_call(
        _final_body,
        grid=(NPAD // COLB,),
        in_specs=[
            pl.BlockSpec((48, COLB), lambda j: (0, j)),
            pl.BlockSpec((1, COLB), lambda j: (0, j)),
            pl.BlockSpec((1, 1, COLB), lambda j: (0, 0, j)),
            pl.BlockSpec((1, 1, COLB), lambda j: (1, 0, j)),
            pl.BlockSpec((48, COLB), lambda j: (0, j)),
            pl.BlockSpec((48, 1), lambda j: (0, 0)),
        ],
        out_specs=pl.BlockSpec((48, COLB), lambda j: (0, j)),
        out_shape=jax.ShapeDtypeStruct((48, NPAD), jnp.float32),
    )(out2T, den2T, ab23, ab23, h2T, b2p)


def _build_a1(att_src1, att_dst1):
    eye = jnp.eye(HEADS, dtype=jnp.float32)
    a_s = (att_src1[:, :, None] * eye[:, None, :]).reshape(D1, HEADS)
    a_d = (att_dst1[:, :, None] * eye[:, None, :]).reshape(D1, HEADS)
    return jnp.concatenate([a_s, a_d], axis=1)


def kernel(x, edge_index, W1, att_src1, att_dst1, b1, W2, att_src2, att_dst2, b2):
    padv = jnp.full((EPAD - E,), N * ((1 << PKS) + 1), dtype=jnp.int32)
    pk = jnp.concatenate(
        [edge_index[1].astype(jnp.int32) * (1 << PKS)
         + edge_index[0].astype(jnp.int32), padv])

    xT = jnp.pad(x.T, ((0, 0), (0, NPAD - N)))
    W1T = W1.T
    A1T = _build_a1(att_src1, att_dst1).T

    hT, aT = _layer1_matmul(xT, W1T, A1T)
    exT, denomP = _k1(aT, pk)
    out1T = _k2(hT, exT, pk)
    actT = _normalize(out1T, denomP, aT, hT, b1)

    W2Tp = jnp.pad(W2.T, ((0, 48 - NCLS), (0, 0)))
    att2p = jnp.pad(jnp.concatenate([att_src2, att_dst2], axis=0),
                    ((0, 0), (0, 48 - NCLS)))
    h2T, ab2T = _layer2_matmul(actT, W2Tp, att2p)
    out2T, den2T = _k3(h2T, ab2T, pk)

    b2p = jnp.pad(b2, (0, 48 - NCLS)).reshape(48, 1)
    outT = _final_logsoftmax(out2T, den2T, ab2T, h2T, b2p)
    return outT[:NCLS, :N].T


# speedup vs baseline: 12.8826x; 1.0003x over previous
import functools
import jax
import jax.numpy as jnp
from jax import lax
from jax.experimental import pallas as pl
from jax.experimental.pallas import tpu as pltpu
from jax.experimental.pallas import tpu_sc as plsc

N = 10000
E = 160000
F_IN = 256
HID = 256
HEADS = 8
NCLS = 40

NPAD = 10240
CH = 4096
EPAD = 163840
D1 = HEADS * HID
COLB = 512
PKS = 14

_MESH = plsc.VectorSubcoreMesh(core_axis_name="c", subcore_axis_name="s")
_SC_PARAMS = pltpu.CompilerParams(needs_layout_passes=False)


def _unpack_edges(v):
    svec = v & jnp.int32((1 << PKS) - 1)
    dvec = lax.shift_right_logical(v, jnp.int32(PKS))
    return svec, dvec


def _mm1_body(xT_ref, w_ref, a_ref, hT_ref, aT_ref):
    h = jnp.dot(w_ref[...], xT_ref[...], preferred_element_type=jnp.float32)
    hT_ref[...] = h
    aT_ref[...] = jnp.dot(a_ref[...], h, preferred_element_type=jnp.float32)


def _layer1_matmul(xT, W1T, A1T):
    return pl.pallas_call(
        _mm1_body,
        grid=(NPAD // COLB,),
        in_specs=[
            pl.BlockSpec((F_IN, COLB), lambda j: (0, j)),
            pl.BlockSpec((D1, F_IN), lambda j: (0, 0)),
            pl.BlockSpec((16, D1), lambda j: (0, 0)),
        ],
        out_specs=[
            pl.BlockSpec((D1, COLB), lambda j: (0, j)),
            pl.BlockSpec((16, COLB), lambda j: (0, j)),
        ],
        out_shape=[
            jax.ShapeDtypeStruct((D1, NPAD), jnp.float32),
            jax.ShapeDtypeStruct((16, NPAD), jnp.float32),
        ],
    )(xT, W1T, A1T)


def _k1_body(aT_hbm, pk_hbm, exT_hbm, denomP_hbm,
             as_tab, ad_tab, dn_tab, pkbuf, exbuf):
    wid = lax.axis_index("s") * 2 + lax.axis_index("c")
    head = wid // 4
    sl = wid % 4
    pltpu.sync_copy(aT_hbm.at[head], as_tab)
    pltpu.sync_copy(aT_hbm.at[8 + head], ad_tab)

    @plsc.parallel_loop(0, NPAD // 16, unroll=8)
    def _zero(j):
        dn_tab[pl.ds(j * 16, 16)] = jnp.zeros((16,), jnp.float32)

    esl = EPAD // 4
    e_base = sl * esl

    def _chunk(ch, _):
        off = e_base + ch * CH
        pltpu.sync_copy(pk_hbm.at[pl.ds(off, CH)], pkbuf)

        @plsc.parallel_loop(0, CH // 16, unroll=8)
        def _iter(i):
            svec, dvec = _unpack_edges(pkbuf[pl.ds(i * 16, 16)])
            z = (plsc.load_gather(as_tab, [svec])
                 + plsc.load_gather(ad_tab, [dvec]))
            ex = jnp.exp(jnp.maximum(z, 0.2 * z))
            exbuf[pl.ds(i * 16, 16)] = ex
            plsc.addupdate_scatter(dn_tab, [dvec], ex)

        pltpu.sync_copy(exbuf, exT_hbm.at[head, pl.ds(off, CH)])
        return 0

    lax.fori_loop(0, esl // CH, _chunk, 0)
    pltpu.sync_copy(dn_tab, denomP_hbm.at[wid])


_k1 = functools.partial(
    pl.kernel,
    out_type=[
        jax.ShapeDtypeStruct((HEADS, EPAD), jnp.float32),
        jax.ShapeDtypeStruct((32, NPAD), jnp.float32),
    ],
    mesh=_MESH,
    compiler_params=_SC_PARAMS,
    scratch_types=[
        pltpu.VMEM((NPAD,), jnp.float32),
        pltpu.VMEM((NPAD,), jnp.float32),
        pltpu.VMEM((NPAD,), jnp.float32),
        pltpu.VMEM((CH,), jnp.int32),
        pltpu.VMEM((CH,), jnp.float32),
    ],
)(_k1_body)


def _k2_body(hT_hbm, exT_hbm, pk_hbm, out1T_hbm,
             h0, h1, h2, h3, a0, a1, a2, a3, pkbuf, ebuf, sem):
    wid = lax.axis_index("s") * 2 + lax.axis_index("c")
    head = wid // 4
    col0 = head * HID + (wid % 4) * 64
    htabs = (h0, h1, h2, h3)
    atabs = (a0, a1, a2, a3)
    nch = EPAD // CH

    def _start(ch):
        off = ch * CH
        so = lax.rem(ch, 2) * CH
        pltpu.async_copy(pk_hbm.at[pl.ds(off, CH)], pkbuf.at[pl.ds(so, CH)], sem)
        pltpu.async_copy(exT_hbm.at[head, pl.ds(off, CH)], ebuf.at[pl.ds(so, CH)], sem)

    def _wait(ch):
        off = ch * CH
        so = lax.rem(ch, 2) * CH
        pltpu.make_async_copy(pk_hbm.at[pl.ds(off, CH)], pkbuf.at[pl.ds(so, CH)], sem).wait()
        pltpu.make_async_copy(exT_hbm.at[head, pl.ds(off, CH)], ebuf.at[pl.ds(so, CH)], sem).wait()

    def _batch(b, _):
        row0 = col0 + b * 4
        for c in range(4):
            pltpu.sync_copy(hT_hbm.at[row0 + c], htabs[c])

        @plsc.parallel_loop(0, NPAD // 16, unroll=8)
        def _zero(j):
            z = jnp.zeros((16,), jnp.float32)
            for c in range(4):
                atabs[c][pl.ds(j * 16, 16)] = z

        _start(0)

        def _chunk(ch, _):
            _wait(ch)

            @pl.when(ch + 1 < nch)
            def _():
                _start(ch + 1)

            so = lax.rem(ch, 2) * CH

            @plsc.parallel_loop(0, CH // 16, unroll=16)
            def _iter(i):
                base = so + i * 16
                svec, dvec = _unpack_edges(pkbuf[pl.ds(base, 16)])
                evec = ebuf[pl.ds(base, 16)]
                for c in range(4):
                    v = plsc.load_gather(htabs[c], [svec])
                    plsc.addupdate_scatter(atabs[c], [dvec], v * evec)

            return 0

        lax.fori_loop(0, nch, _chunk, 0)
        for c in range(4):
            pltpu.sync_copy(atabs[c], out1T_hbm.at[row0 + c])
        return 0

    lax.fori_loop(0, 16, _batch, 0)


_k2 = functools.partial(
    pl.kernel,
    out_type=jax.ShapeDtypeStruct((D1, NPAD), jnp.float32),
    mesh=_MESH,
    compiler_params=_SC_PARAMS,
    scratch_types=(
        [pltpu.VMEM((NPAD,), jnp.float32)] * 8
        + [pltpu.VMEM((2 * CH,), jnp.int32),
           pltpu.VMEM((2 * CH,), jnp.float32),
           pltpu.SemaphoreType.DMA]
    ),
)(_k2_body)


def _norm_body(o_ref, d_ref, as_ref, ad_ref, h_ref, b_ref, act_ref):
    za = as_ref[0] + ad_ref[0]
    ex_self = jnp.exp(jnp.maximum(za, 0.2 * za))
    dsum = jnp.sum(d_ref[0], axis=0, keepdims=True) + ex_self
    z = (o_ref[...] + ex_self * h_ref[...]) / (dsum + 1e-16) + b_ref[...]
    act_ref[...] = jnp.where(z > 0, z, jnp.exp(jnp.minimum(z, 0.0)) - 1.0)


def _normalize(out1T, denomP, aT, hT, b1):
    aT3 = aT.reshape(16, 1, NPAD)
    return pl.pallas_call(
        _norm_body,
        grid=(HEADS, NPAD // COLB),
        in_specs=[
            pl.BlockSpec((HID, COLB), lambda k, j: (k, j)),
            pl.BlockSpec((1, 4, COLB), lambda k, j: (k, 0, j)),
            pl.BlockSpec((1, 1, COLB), lambda k, j: (k, 0, j)),
            pl.BlockSpec((1, 1, COLB), lambda k, j: (8 + k, 0, j)),
            pl.BlockSpec((HID, COLB), lambda k, j: (k, j)),
            pl.BlockSpec((HID, 1), lambda k, j: (k, 0)),
        ],
        out_specs=pl.BlockSpec((HID, COLB), lambda k, j: (k, j)),
        out_shape=jax.ShapeDtypeStruct((D1, NPAD), jnp.float32),
    )(out1T, denomP.reshape(HEADS, 4, NPAD), aT3, aT3, hT, b1.reshape(D1, 1))


def _mm2_body(act_ref, w_ref, a_ref, h2_ref, ab_ref):
    h2 = jnp.dot(w_ref[...], act_ref[...], preferred_element_type=jnp.float32)
    h2_ref[...] = h2
    ab_ref[...] = jnp.dot(a_ref[...], h2, preferred_element_type=jnp.float32)


def _layer2_matmul(actT, W2Tp, att2p):
    return pl.pallas_call(
        _mm2_body,
        grid=(NPAD // COLB,),
        in_specs=[
            pl.BlockSpec((D1, COLB), lambda j: (0, j)),
            pl.BlockSpec((48, D1), lambda j: (0, 0)),
            pl.BlockSpec((2, 48), lambda j: (0, 0)),
        ],
        out_specs=[
            pl.BlockSpec((48, COLB), lambda j: (0, j)),
            pl.BlockSpec((2, COLB), lambda j: (0, j)),
        ],
        out_shape=[
            jax.ShapeDtypeStruct((48, NPAD), jnp.float32),
            jax.ShapeDtypeStruct((2, NPAD), jnp.float32),
        ],
    )(actT, W2Tp, att2p)


def _k3_body(h2T_hbm, ab2T_hbm, pk_hbm, out2T_hbm, den2_hbm,
             as_tab, ad_tab, t0, t1, a0, a1, dn, pkbuf):
    wid = lax.axis_index("s") * 2 + lax.axis_index("c")

    @pl.when(wid <= 24)
    def _():
        pltpu.sync_copy(ab2T_hbm.at[0], as_tab)
        pltpu.sync_copy(ab2T_hbm.at[1], ad_tab)
        c0 = wid * 2

        @pl.when(wid < 24)
        def _():
            pltpu.sync_copy(h2T_hbm.at[c0], t0)
            pltpu.sync_copy(h2T_hbm.at[c0 + 1], t1)

        @plsc.parallel_loop(0, NPAD // 16, unroll=8)
        def _zero(j):
            z = jnp.zeros((16,), jnp.float32)
            a0[pl.ds(j * 16, 16)] = z
            a1[pl.ds(j * 16, 16)] = z
            dn[pl.ds(j * 16, 16)] = z

        def _chunk(ch, _):
            off = ch * CH
            pltpu.sync_copy(pk_hbm.at[pl.ds(off, CH)], pkbuf)

            @plsc.parallel_loop(0, CH // 16, unroll=8)
            def _iter(i):
                svec, dvec = _unpack_edges(pkbuf[pl.ds(i * 16, 16)])
                z = (plsc.load_gather(as_tab, [svec])
                     + plsc.load_gather(ad_tab, [dvec]))
                ex = jnp.exp(jnp.maximum(z, 0.2 * z))

                @pl.when(wid < 24)
                def _():
                    for tt, aa in ((t0, a0), (t1, a1)):
                        v = plsc.load_gather(tt, [svec])
                        plsc.addupdate_scatter(aa, [dvec], v * ex)

                @pl.when(wid == 24)
                def _():
                    plsc.addupdate_scatter(dn, [dvec], ex)

            return 0

        lax.fori_loop(0, EPAD // CH, _chunk, 0)

        @pl.when(wid < 24)
        def _():
            pltpu.sync_copy(a0, out2T_hbm.at[c0])
            pltpu.sync_copy(a1, out2T_hbm.at[c0 + 1])

        @pl.when(wid == 24)
        def _():
            pltpu.sync_copy(dn, den2_hbm.at[0])


_k3 = functools.partial(
    pl.kernel,
    out_type=[
        jax.ShapeDtypeStruct((48, NPAD), jnp.float32),
        jax.ShapeDtypeStruct((1, NPAD), jnp.float32),
    ],
    mesh=_MESH,
    compiler_params=_SC_PARAMS,
    scratch_types=(
        [pltpu.VMEM((NPAD,), jnp.float32)] * 7
        + [pltpu.VMEM((CH,), jnp.int32)]
    ),
)(_k3_body)


def _final_body(o_ref, d_ref, as_ref, ad_ref, h_ref, b_ref, out_ref):
    za = as_ref[0] + ad_ref[0]
    ex_self = jnp.exp(jnp.maximum(za, 0.2 * za))
    num = o_ref[...] + ex_self * h_ref[...]
    den = d_ref[...] + ex_self + 1e-16
    z = num / den + b_ref[...]
    rowid = lax.broadcasted_iota(jnp.int32, (48, COLB), 0)
    valid = rowid < NCLS
    zm = jnp.where(valid, z, -1e30)
    m = jnp.max(zm, axis=0, keepdims=True)
    s = jnp.sum(jnp.where(valid, jnp.exp(z - m), 0.0), axis=0, keepdims=True)
    out_ref[...] = z - m - jnp.log(s)


def _final_logsoftmax(out2T, den2T, ab2T, h2T, b2p):
    ab23 = ab2T.reshape(2, 1, NPAD)
    return pl.pallas_call(
        _final_body,
        grid=(NPAD // COLB,),
        in_specs=[
            pl.BlockSpec((48, COLB), lambda j: (0, j)),
            pl.BlockSpec((1, COLB), lambda j: (0, j)),
            pl.BlockSpec((1, 1, COLB), lambda j: (0, 0, j)),
            pl.BlockSpec((1, 1, COLB), lambda j: (1, 0, j)),
            pl.BlockSpec((48, COLB), lambda j: (0, j)),
            pl.BlockSpec((48, 1), lambda j: (0, 0)),
        ],
        out_specs=pl.BlockSpec((48, COLB), lambda j: (0, j)),
        out_shape=jax.ShapeDtypeStruct((48, NPAD), jnp.float32),
    )(out2T, den2T, ab23, ab23, h2T, b2p)


def _build_a1(att_src1, att_dst1):
    eye = jnp.eye(HEADS, dtype=jnp.float32)
    a_s = (att_src1[:, :, None] * eye[:, None, :]).reshape(D1, HEADS)
    a_d = (att_dst1[:, :, None] * eye[:, None, :]).reshape(D1, HEADS)
    return jnp.concatenate([a_s, a_d], axis=1)


def kernel(x, edge_index, W1, att_src1, att_dst1, b1, W2, att_src2, att_dst2, b2):
    padv = jnp.full((EPAD - E,), N * ((1 << PKS) + 1), dtype=jnp.int32)
    pk = jnp.concatenate(
        [edge_index[1].astype(jnp.int32) * (1 << PKS)
         + edge_index[0].astype(jnp.int32), padv])

    xT = jnp.pad(x.T, ((0, 0), (0, NPAD - N)))
    W1T = W1.T
    A1T = _build_a1(att_src1, att_dst1).T

    hT, aT = _layer1_matmul(xT, W1T, A1T)
    exT, denomP = _k1(aT, pk)
    out1T = _k2(hT, exT, pk)
    actT = _normalize(out1T, denomP, aT, hT, b1)

    W2Tp = jnp.pad(W2.T, ((0, 48 - NCLS), (0, 0)))
    att2p = jnp.pad(jnp.concatenate([att_src2, att_dst2], axis=0),
                    ((0, 0), (0, 48 - NCLS)))
    h2T, ab2T = _layer2_matmul(actT, W2Tp, att2p)
    out2T, den2T = _k3(h2T, ab2T, pk)

    b2p = jnp.pad(b2, (0, 48 - NCLS)).reshape(48, 1)
    outT = _final_logsoftmax(out2T, den2T, ab2T, h2T, b2p)
    return outT[:NCLS, :N].T


# speedup vs baseline: 14.1762x; 1.1004x over previous
import functools
import jax
import jax.numpy as jnp
from jax import lax
from jax.experimental import pallas as pl
from jax.experimental.pallas import tpu as pltpu
from jax.experimental.pallas import tpu_sc as plsc

N = 10000
E = 160000
F_IN = 256
HID = 256
HEADS = 8
NCLS = 40

NPAD = 10240
CH = 4096
EPAD = 163840
D1 = HEADS * HID
COLB = 512
PKS = 14

_MESH = plsc.VectorSubcoreMesh(core_axis_name="c", subcore_axis_name="s")
_SC_PARAMS = pltpu.CompilerParams(needs_layout_passes=False)


def _unpack_edges(v):
    svec = v & jnp.int32((1 << PKS) - 1)
    dvec = lax.shift_right_logical(v, jnp.int32(PKS))
    return svec, dvec


def _mm1_body(xT_ref, w_ref, a_ref, hT_ref, aT_ref, hTp_ref):
    h = jnp.dot(w_ref[...], xT_ref[...], preferred_element_type=jnp.float32)
    hT_ref[...] = h
    aT_ref[...] = jnp.dot(a_ref[...], h, preferred_element_type=jnp.float32)
    hb = lax.bitcast_convert_type(h.astype(jnp.bfloat16), jnp.uint16)
    hb = hb.astype(jnp.uint32).reshape(D1 // 2, 2, COLB)
    word = hb[:, 0, :] | (hb[:, 1, :] << 16)
    hTp_ref[...] = lax.bitcast_convert_type(word, jnp.int32)


def _layer1_matmul(xT, W1T, A1T):
    return pl.pallas_call(
        _mm1_body,
        grid=(NPAD // COLB,),
        in_specs=[
            pl.BlockSpec((F_IN, COLB), lambda j: (0, j)),
            pl.BlockSpec((D1, F_IN), lambda j: (0, 0)),
            pl.BlockSpec((16, D1), lambda j: (0, 0)),
        ],
        out_specs=[
            pl.BlockSpec((D1, COLB), lambda j: (0, j)),
            pl.BlockSpec((16, COLB), lambda j: (0, j)),
            pl.BlockSpec((D1 // 2, COLB), lambda j: (0, j)),
        ],
        out_shape=[
            jax.ShapeDtypeStruct((D1, NPAD), jnp.float32),
            jax.ShapeDtypeStruct((16, NPAD), jnp.float32),
            jax.ShapeDtypeStruct((D1 // 2, NPAD), jnp.int32),
        ],
    )(xT, W1T, A1T)


def _k1_body(aT_hbm, pk_hbm, exT_hbm, denomP_hbm,
             as_tab, ad_tab, dn_tab, pkbuf, exbuf):
    wid = lax.axis_index("s") * 2 + lax.axis_index("c")
    head = wid // 4
    sl = wid % 4
    pltpu.sync_copy(aT_hbm.at[head], as_tab)
    pltpu.sync_copy(aT_hbm.at[8 + head], ad_tab)

    @plsc.parallel_loop(0, NPAD // 16, unroll=8)
    def _zero(j):
        dn_tab[pl.ds(j * 16, 16)] = jnp.zeros((16,), jnp.float32)

    esl = EPAD // 4
    e_base = sl * esl

    def _chunk(ch, _):
        off = e_base + ch * CH
        pltpu.sync_copy(pk_hbm.at[pl.ds(off, CH)], pkbuf)

        @plsc.parallel_loop(0, CH // 16, unroll=8)
        def _iter(i):
            svec, dvec = _unpack_edges(pkbuf[pl.ds(i * 16, 16)])
            z = (plsc.load_gather(as_tab, [svec])
                 + plsc.load_gather(ad_tab, [dvec]))
            ex = jnp.exp(jnp.maximum(z, 0.2 * z))
            exbuf[pl.ds(i * 16, 16)] = ex
            plsc.addupdate_scatter(dn_tab, [dvec], ex)

        pltpu.sync_copy(exbuf, exT_hbm.at[head, pl.ds(off, CH)])
        return 0

    lax.fori_loop(0, esl // CH, _chunk, 0)
    pltpu.sync_copy(dn_tab, denomP_hbm.at[wid])


_k1 = functools.partial(
    pl.kernel,
    out_type=[
        jax.ShapeDtypeStruct((HEADS, EPAD), jnp.float32),
        jax.ShapeDtypeStruct((32, NPAD), jnp.float32),
    ],
    mesh=_MESH,
    compiler_params=_SC_PARAMS,
    scratch_types=[
        pltpu.VMEM((NPAD,), jnp.float32),
        pltpu.VMEM((NPAD,), jnp.float32),
        pltpu.VMEM((NPAD,), jnp.float32),
        pltpu.VMEM((CH,), jnp.int32),
        pltpu.VMEM((CH,), jnp.float32),
    ],
)(_k1_body)


def _k2_body(hTp_hbm, exT_hbm, pk_hbm, out1T_hbm,
             p0t, p1t, a0, a1, a2, a3, pkbuf, ebuf, sem):
    wid = lax.axis_index("s") * 2 + lax.axis_index("c")
    head = wid // 4
    col0 = head * HID + (wid % 4) * 64
    ptabs = (p0t, p1t)
    atabs = (a0, a1, a2, a3)
    nch = EPAD // CH

    def _start(ch):
        off = ch * CH
        so = lax.rem(ch, 2) * CH
        pltpu.async_copy(pk_hbm.at[pl.ds(off, CH)], pkbuf.at[pl.ds(so, CH)], sem)
        pltpu.async_copy(exT_hbm.at[head, pl.ds(off, CH)], ebuf.at[pl.ds(so, CH)], sem)

    def _wait(ch):
        off = ch * CH
        so = lax.rem(ch, 2) * CH
        pltpu.make_async_copy(pk_hbm.at[pl.ds(off, CH)], pkbuf.at[pl.ds(so, CH)], sem).wait()
        pltpu.make_async_copy(exT_hbm.at[head, pl.ds(off, CH)], ebuf.at[pl.ds(so, CH)], sem).wait()

    def _batch(b, _):
        row0 = col0 + b * 4
        for p in range(2):
            pltpu.sync_copy(hTp_hbm.at[col0 // 2 + b * 2 + p], ptabs[p])

        @plsc.parallel_loop(0, NPAD // 16, unroll=8)
        def _zero(j):
            z = jnp.zeros((16,), jnp.float32)
            for c in range(4):
                atabs[c][pl.ds(j * 16, 16)] = z

        _start(0)

        def _chunk(ch, _):
            _wait(ch)

            @pl.when(ch + 1 < nch)
            def _():
                _start(ch + 1)

            so = lax.rem(ch, 2) * CH

            @plsc.parallel_loop(0, CH // 16, unroll=16)
            def _iter(i):
                base = so + i * 16
                svec, dvec = _unpack_edges(pkbuf[pl.ds(base, 16)])
                evec = ebuf[pl.ds(base, 16)]
                for p in range(2):
                    w = plsc.load_gather(ptabs[p], [svec])
                    lo = plsc.bitcast(lax.shift_left(w, 16), jnp.float32)
                    hi = plsc.bitcast(w & jnp.int32(-65536), jnp.float32)
                    plsc.addupdate_scatter(atabs[2 * p], [dvec], lo * evec)
                    plsc.addupdate_scatter(atabs[2 * p + 1], [dvec], hi * evec)

            return 0

        lax.fori_loop(0, nch, _chunk, 0)
        for c in range(4):
            pltpu.sync_copy(atabs[c], out1T_hbm.at[row0 + c])
        return 0

    lax.fori_loop(0, 16, _batch, 0)


_k2 = functools.partial(
    pl.kernel,
    out_type=jax.ShapeDtypeStruct((D1, NPAD), jnp.float32),
    mesh=_MESH,
    compiler_params=_SC_PARAMS,
    scratch_types=(
        [pltpu.VMEM((NPAD,), jnp.int32)] * 2
        + [pltpu.VMEM((NPAD,), jnp.float32)] * 4
        + [pltpu.VMEM((2 * CH,), jnp.int32),
           pltpu.VMEM((2 * CH,), jnp.float32),
           pltpu.SemaphoreType.DMA]
    ),
)(_k2_body)


def _norm_body(o_ref, d_ref, as_ref, ad_ref, h_ref, b_ref, act_ref):
    za = as_ref[0] + ad_ref[0]
    ex_self = jnp.exp(jnp.maximum(za, 0.2 * za))
    dsum = jnp.sum(d_ref[0], axis=0, keepdims=True) + ex_self
    z = (o_ref[...] + ex_self * h_ref[...]) / (dsum + 1e-16) + b_ref[...]
    act_ref[...] = jnp.where(z > 0, z, jnp.exp(jnp.minimum(z, 0.0)) - 1.0)


def _normalize(out1T, denomP, aT, hT, b1):
    aT3 = aT.reshape(16, 1, NPAD)
    return pl.pallas_call(
        _norm_body,
        grid=(HEADS, NPAD // COLB),
        in_specs=[
            pl.BlockSpec((HID, COLB), lambda k, j: (k, j)),
            pl.BlockSpec((1, 4, COLB), lambda k, j: (k, 0, j)),
            pl.BlockSpec((1, 1, COLB), lambda k, j: (k, 0, j)),
            pl.BlockSpec((1, 1, COLB), lambda k, j: (8 + k, 0, j)),
            pl.BlockSpec((HID, COLB), lambda k, j: (k, j)),
            pl.BlockSpec((HID, 1), lambda k, j: (k, 0)),
        ],
        out_specs=pl.BlockSpec((HID, COLB), lambda k, j: (k, j)),
        out_shape=jax.ShapeDtypeStruct((D1, NPAD), jnp.float32),
    )(out1T, denomP.reshape(HEADS, 4, NPAD), aT3, aT3, hT, b1.reshape(D1, 1))


def _mm2_body(act_ref, w_ref, a_ref, h2_ref, ab_ref):
    h2 = jnp.dot(w_ref[...], act_ref[...], preferred_element_type=jnp.float32)
    h2_ref[...] = h2
    ab_ref[...] = jnp.dot(a_ref[...], h2, preferred_element_type=jnp.float32)


def _layer2_matmul(actT, W2Tp, att2p):
    return pl.pallas_call(
        _mm2_body,
        grid=(NPAD // COLB,),
        in_specs=[
            pl.BlockSpec((D1, COLB), lambda j: (0, j)),
            pl.BlockSpec((48, D1), lambda j: (0, 0)),
            pl.BlockSpec((2, 48), lambda j: (0, 0)),
        ],
        out_specs=[
            pl.BlockSpec((48, COLB), lambda j: (0, j)),
            pl.BlockSpec((2, COLB), lambda j: (0, j)),
        ],
        out_shape=[
            jax.ShapeDtypeStruct((48, NPAD), jnp.float32),
            jax.ShapeDtypeStruct((2, NPAD), jnp.float32),
        ],
    )(actT, W2Tp, att2p)


def _k3_body(h2T_hbm, ab2T_hbm, pk_hbm, out2T_hbm, den2_hbm,
             as_tab, ad_tab, t0, t1, a0, a1, dn, pkbuf):
    wid = lax.axis_index("s") * 2 + lax.axis_index("c")

    @pl.when(wid <= 24)
    def _():
        pltpu.sync_copy(ab2T_hbm.at[0], as_tab)
        pltpu.sync_copy(ab2T_hbm.at[1], ad_tab)
        c0 = wid * 2

        @pl.when(wid < 24)
        def _():
            pltpu.sync_copy(h2T_hbm.at[c0], t0)
            pltpu.sync_copy(h2T_hbm.at[c0 + 1], t1)

        @plsc.parallel_loop(0, NPAD // 16, unroll=8)
        def _zero(j):
            z = jnp.zeros((16,), jnp.float32)
            a0[pl.ds(j * 16, 16)] = z
            a1[pl.ds(j * 16, 16)] = z
            dn[pl.ds(j * 16, 16)] = z

        def _chunk(ch, _):
            off = ch * CH
            pltpu.sync_copy(pk_hbm.at[pl.ds(off, CH)], pkbuf)

            @plsc.parallel_loop(0, CH // 16, unroll=8)
            def _iter(i):
                svec, dvec = _unpack_edges(pkbuf[pl.ds(i * 16, 16)])
                z = (plsc.load_gather(as_tab, [svec])
                     + plsc.load_gather(ad_tab, [dvec]))
                ex = jnp.exp(jnp.maximum(z, 0.2 * z))

                @pl.when(wid < 24)
                def _():
                    for tt, aa in ((t0, a0), (t1, a1)):
                        v = plsc.load_gather(tt, [svec])
                        plsc.addupdate_scatter(aa, [dvec], v * ex)

                @pl.when(wid == 24)
                def _():
                    plsc.addupdate_scatter(dn, [dvec], ex)

            return 0

        lax.fori_loop(0, EPAD // CH, _chunk, 0)

        @pl.when(wid < 24)
        def _():
            pltpu.sync_copy(a0, out2T_hbm.at[c0])
            pltpu.sync_copy(a1, out2T_hbm.at[c0 + 1])

        @pl.when(wid == 24)
        def _():
            pltpu.sync_copy(dn, den2_hbm.at[0])


_k3 = functools.partial(
    pl.kernel,
    out_type=[
        jax.ShapeDtypeStruct((48, NPAD), jnp.float32),
        jax.ShapeDtypeStruct((1, NPAD), jnp.float32),
    ],
    mesh=_MESH,
    compiler_params=_SC_PARAMS,
    scratch_types=(
        [pltpu.VMEM((NPAD,), jnp.float32)] * 7
        + [pltpu.VMEM((CH,), jnp.int32)]
    ),
)(_k3_body)


def _final_body(o_ref, d_ref, as_ref, ad_ref, h_ref, b_ref, out_ref):
    za = as_ref[0] + ad_ref[0]
    ex_self = jnp.exp(jnp.maximum(za, 0.2 * za))
    num = o_ref[...] + ex_self * h_ref[...]
    den = d_ref[...] + ex_self + 1e-16
    z = num / den + b_ref[...]
    rowid = lax.broadcasted_iota(jnp.int32, (48, COLB), 0)
    valid = rowid < NCLS
    zm = jnp.where(valid, z, -1e30)
    m = jnp.max(zm, axis=0, keepdims=True)
    s = jnp.sum(jnp.where(valid, jnp.exp(z - m), 0.0), axis=0, keepdims=True)
    out_ref[...] = z - m - jnp.log(s)


def _final_logsoftmax(out2T, den2T, ab2T, h2T, b2p):
    ab23 = ab2T.reshape(2, 1, NPAD)
    return pl.pallas_call(
        _final_body,
        grid=(NPAD // COLB,),
        in_specs=[
            pl.BlockSpec((48, COLB), lambda j: (0, j)),
            pl.BlockSpec((1, COLB), lambda j: (0, j)),
            pl.BlockSpec((1, 1, COLB), lambda j: (0, 0, j)),
            pl.BlockSpec((1, 1, COLB), lambda j: (1, 0, j)),
            pl.BlockSpec((48, COLB), lambda j: (0, j)),
            pl.BlockSpec((48, 1), lambda j: (0, 0)),
        ],
        out_specs=pl.BlockSpec((48, COLB), lambda j: (0, j)),
        out_shape=jax.ShapeDtypeStruct((48, NPAD), jnp.float32),
    )(out2T, den2T, ab23, ab23, h2T, b2p)


def _build_a1(att_src1, att_dst1):
    eye = jnp.eye(HEADS, dtype=jnp.float32)
    a_s = (att_src1[:, :, None] * eye[:, None, :]).reshape(D1, HEADS)
    a_d = (att_dst1[:, :, None] * eye[:, None, :]).reshape(D1, HEADS)
    return jnp.concatenate([a_s, a_d], axis=1)


def kernel(x, edge_index, W1, att_src1, att_dst1, b1, W2, att_src2, att_dst2, b2):
    padv = jnp.full((EPAD - E,), N * ((1 << PKS) + 1), dtype=jnp.int32)
    pk = jnp.concatenate(
        [edge_index[1].astype(jnp.int32) * (1 << PKS)
         + edge_index[0].astype(jnp.int32), padv])

    xT = jnp.pad(x.T, ((0, 0), (0, NPAD - N)))
    W1T = W1.T
    A1T = _build_a1(att_src1, att_dst1).T

    hT, aT, hTp = _layer1_matmul(xT, W1T, A1T)
    exT, denomP = _k1(aT, pk)
    out1T = _k2(hTp, exT, pk)
    actT = _normalize(out1T, denomP, aT, hT, b1)

    W2Tp = jnp.pad(W2.T, ((0, 48 - NCLS), (0, 0)))
    att2p = jnp.pad(jnp.concatenate([att_src2, att_dst2], axis=0),
                    ((0, 0), (0, 48 - NCLS)))
    h2T, ab2T = _layer2_matmul(actT, W2Tp, att2p)
    out2T, den2T = _k3(h2T, ab2T, pk)

    b2p = jnp.pad(b2, (0, 48 - NCLS)).reshape(48, 1)
    outT = _final_logsoftmax(out2T, den2T, ab2T, h2T, b2p)
    return outT[:NCLS, :N].T
